# baseline (speedup 1.0000x reference)
import functools

import numpy as np
import jax
import jax.numpy as jnp
from jax import lax
from jax.experimental import pallas as pl
from jax.experimental.pallas import tpu as pltpu
from jax.experimental.pallas import tpu_sc as plsc

L = 16
F = 2
T = 2 ** 20
MASK = T - 1
PI2_I32 = 2654435761 - 2 ** 32
NL = [16, 20, 25, 32, 40, 50, 64, 80, 101, 128, 161, 203, 256, 322, 406, 512]

B = 131072
NW = 32
PW = B // NW
C = 16
G = PW // C

NCOARSE = 10
NFINE = L - NCOARSE
ROWS = C * NFINE * 4 * 2
NDMA = ROWS // 128

W_LVL = [NL[l] + 1 for l in range(NCOARSE)]
SH = []
PST = []
_cur = 0
for _l in range(NCOARSE):
    _cells = W_LVL[_l] * W_LVL[_l]
    _s = 16 * ((_cells + 16 * 16 - 1) // (16 * 16))
    SH.append(_s)
    PST.append(_cur)
    _cur += 16 * _s
GRID_CELLS = _cur
GRID_WORDS = GRID_CELLS * 2
SH_MAX = max(SH)
BUILD_GROUP = ROWS // 32

_SC_PARAMS = pltpu.CompilerParams(
    needs_layout_passes=False, use_tc_tiling_on_sc=False
)
_MESH = dict(core_axis_name="c", subcore_axis_name="s", num_cores=2,
             num_subcores=16)


def _iota16():
    return lax.iota(jnp.int32, 16)


def _hash(ia, ib):
    return (ia ^ (ib * PI2_I32)) & MASK


def _build_grid(tab_hbm, sid, gridv, gridsh, rows, staging, sem):
    lanes = _iota16()
    for l in range(NCOARSE):
        w = W_LVL[l]
        cells = w * w
        cbase = sid * np.int32(SH[l])
        nblk = SH[l] // 16
        for blk0 in range(0, nblk, BUILD_GROUP):
            blks = range(blk0, min(blk0 + BUILD_GROUP, nblk))
            copies = []
            for blk in blks:
                c = cbase + (lanes + np.int32(blk * 16))
                c = jnp.minimum(c, np.int32(cells - 1))
                i = c // w
                j = c - i * w
                h2 = _hash(i, j) * 2
                copies.append(
                    pltpu.async_copy(
                        tab_hbm.at[h2],
                        rows.at[pl.ds((blk - blk0) * 32, 16)], sem)
                )
                copies.append(
                    pltpu.async_copy(
                        tab_hbm.at[h2 + 1],
                        rows.at[pl.ds((blk - blk0) * 32 + 16, 16)], sem)
                )
            for cp in copies:
                cp.wait()
            for blk in blks:
                ridx = lanes + np.int32((blk - blk0) * 32)
                lc = jnp.full((16,), l, jnp.int32)
                f0 = plsc.load_gather(rows, [ridx, lc])
                f1 = plsc.load_gather(rows, [ridx + 16, lc])
                sidx = (lanes + np.int32(blk * 16)) * 2
                plsc.store_scatter(staging, [sidx], f0)
                plsc.store_scatter(staging, [sidx + 1], f1)
        dst = (np.int32(PST[l]) + cbase) * 2
        pltpu.sync_copy(staging.at[pl.ds(0, SH[l] * 2)],
                        gridsh.at[pl.ds(dst, SH[l] * 2)])
    plsc.subcore_barrier()
    pltpu.sync_copy(gridsh, gridv)


def _main_body(x_hbm, tab_hbm, out_hbm, xv, gridv, gridsh, idxm, rows, outv,
               staging, sem):
    cid = lax.axis_index("c")
    sid = lax.axis_index("s")
    wid = sid * 2 + cid
    lanes = _iota16()
    zeros = jnp.full((16,), 0, jnp.int32)

    _build_grid(tab_hbm, sid, gridv, gridsh, rows, staging, sem)

    def chunk(g, _):
        base = wid * np.int32(PW) + g * np.int32(C)
        pltpu.sync_copy(x_hbm.at[pl.ds(base, C)], xv)

        for k in range(C // 16):
            pidx = lanes + np.int32(k * 16)
            xa = plsc.load_gather(xv, [pidx, zeros])
            xb = plsc.load_gather(xv, [pidx, zeros + 1])
            for li, l in enumerate(range(NCOARSE, L)):
                nl = jnp.float32(NL[l])
                ia = (xa * nl).astype(jnp.int32)
                ib = (xb * nl).astype(jnp.int32)
                ia1 = ia + 1
                t0 = ib * PI2_I32
                t1 = (ib + 1) * PI2_I32
                hs = (
                    (ia ^ t0) & MASK,
                    (ia1 ^ t0) & MASK,
                    (ia ^ t1) & MASK,
                    (ia1 ^ t1) & MASK,
                )
                for c in range(4):
                    fp = ((k * NFINE + li) * 4 + c) * 32
                    h2 = hs[c] * 2
                    idxm[fp // 128, pl.ds(fp % 128, 16)] = h2
                    fp += 16
                    idxm[fp // 128, pl.ds(fp % 128, 16)] = h2 + 1

        copies = [
            pltpu.async_copy(
                tab_hbm.at[idxm.at[j]], rows.at[pl.ds(j * 128, 128)], sem
            )
            for j in range(NDMA)
        ]

        for k in range(C // 16):
            pidx = lanes + np.int32(k * 16)
            xa = plsc.load_gather(xv, [pidx, zeros])
            xb = plsc.load_gather(xv, [pidx, zeros + 1])
            for l in range(NCOARSE):
                nl = jnp.float32(NL[l])
                w = np.int32(W_LVL[l])
                wa = xa * nl
                wb = xb * nl
                ia = wa.astype(jnp.int32)
                ib = wb.astype(jnp.int32)
                fa = wa - ia.astype(jnp.float32)
                fb = wb - ib.astype(jnp.float32)
                ga = 1.0 - fa
                gb = 1.0 - fb
                w00 = ga * gb
                w10 = fa * gb
                w01 = ga * fb
                w11 = fa * fb
                c00 = (ia * w + ib + np.int32(PST[l])) * 2
                c10 = c00 + 2 * w
                h00f0 = plsc.load_gather(gridv, [c00])
                h00f1 = plsc.load_gather(gridv, [c00 + 1])
                h10f0 = plsc.load_gather(gridv, [c10])
                h10f1 = plsc.load_gather(gridv, [c10 + 1])
                h01f0 = plsc.load_gather(gridv, [c00 + 2])
                h01f1 = plsc.load_gather(gridv, [c00 + 3])
                h11f0 = plsc.load_gather(gridv, [c10 + 2])
                h11f1 = plsc.load_gather(gridv, [c10 + 3])
                y0 = w00 * h00f0 + w10 * h10f0 + w01 * h01f0 + w11 * h11f0
                y1 = w00 * h00f1 + w10 * h10f1 + w01 * h01f1 + w11 * h11f1
                cc = jnp.full((16,), 2 * l, jnp.int32)
                plsc.store_scatter(outv, [pidx, cc], y0)
                plsc.store_scatter(outv, [pidx, cc + 1], y1)

        for cp in copies:
            cp.wait()

        for k in range(C // 16):
            pidx = lanes + np.int32(k * 16)
            xa = plsc.load_gather(xv, [pidx, zeros])
            xb = plsc.load_gather(xv, [pidx, zeros + 1])
            for li, l in enumerate(range(NCOARSE, L)):
                nl = jnp.float32(NL[l])
                wa = xa * nl
                wb = xb * nl
                ia = wa.astype(jnp.int32)
                ib = wb.astype(jnp.int32)
                fa = wa - ia.astype(jnp.float32)
                fb = wb - ib.astype(jnp.float32)
                ga = 1.0 - fa
                gb = 1.0 - fb
                w00 = ga * gb
                w10 = fa * gb
                w01 = ga * fb
                w11 = fa * fb
                rbase = (k * NFINE + li) * 128
                lc = jnp.full((16,), l, jnp.int32)
                h = []
                for c in range(4):
                    ridx = lanes + np.int32(rbase + c * 32)
                    h.append(plsc.load_gather(rows, [ridx, lc]))
                    h.append(plsc.load_gather(rows, [ridx + 16, lc]))
                y0 = w00 * h[0] + w10 * h[2] + w01 * h[4] + w11 * h[6]
                y1 = w00 * h[1] + w10 * h[3] + w01 * h[5] + w11 * h[7]
                cc = jnp.full((16,), 2 * l, jnp.int32)
                plsc.store_scatter(outv, [pidx, cc], y0)
                plsc.store_scatter(outv, [pidx, cc + 1], y1)

        pltpu.sync_copy(outv, out_hbm.at[pl.ds(base, C), :])
        return ()

    lax.fori_loop(np.int32(0), np.int32(G), chunk, (), unroll=False)


@jax.jit
def _kernel_impl(x, hash_table):
    tab = hash_table.reshape(T * F, L)
    run = pl.kernel(
        _main_body,
        out_type=jax.ShapeDtypeStruct((B, F * L), jnp.float32),
        mesh=plsc.VectorSubcoreMesh(**_MESH),
        scratch_types=[
            pltpu.VMEM((C, 2), jnp.float32),
            pltpu.VMEM((GRID_WORDS,), jnp.float32),
            pltpu.VMEM_SHARED((GRID_WORDS,), jnp.float32),
            pltpu.VMEM((NDMA, 128), jnp.int32),
            pltpu.VMEM((ROWS, L), jnp.float32),
            pltpu.VMEM((C, F * L), jnp.float32),
            pltpu.VMEM((SH_MAX * 2,), jnp.float32),
            pltpu.SemaphoreType.DMA,
        ],
        compiler_params=_SC_PARAMS,
    )
    return run(x, tab)


def kernel(x, hash_table):
    prev = jax.config.jax_enable_x64
    if prev:
        jax.config.update("jax_enable_x64", False)
    try:
        return _kernel_impl(x, hash_table)
    finally:
        if prev:
            jax.config.update("jax_enable_x64", True)


# speedup vs baseline: 1.5727x; 1.5727x over previous
import functools

import numpy as np
import jax
import jax.numpy as jnp
from jax import lax
from jax.experimental import pallas as pl
from jax.experimental.pallas import tpu as pltpu
from jax.experimental.pallas import tpu_sc as plsc

L = 16
F = 2
T = 2 ** 20
MASK = T - 1
PI2_I32 = 2654435761 - 2 ** 32
NL = [16, 20, 25, 32, 40, 50, 64, 80, 101, 128, 161, 203, 256, 322, 406, 512]

B = 131072
NW = 32
PW = B // NW
C = 32
G = PW // C

NCOARSE = 10
NFINE = L - NCOARSE
ROWS = C * NFINE * 4
NDMA = ROWS // 128

W_LVL = [NL[l] + 1 for l in range(NCOARSE)]
SH = []
PST = []
_cur = 0
for _l in range(NCOARSE):
    _cells = W_LVL[_l] * W_LVL[_l]
    _s = 16 * ((_cells + 32 * 16 - 1) // (32 * 16))
    SH.append(_s)
    PST.append(_cur)
    _cur += 32 * _s
GRID_CELLS = _cur
GRID_WORDS = GRID_CELLS * 2
SH_MAX = max(SH)

_SC_PARAMS = pltpu.CompilerParams(
    needs_layout_passes=False, use_tc_tiling_on_sc=False
)
_MESH = dict(core_axis_name="c", subcore_axis_name="s", num_cores=2,
             num_subcores=16)


def _iota16():
    return lax.iota(jnp.int32, 16)


def _hash(ia, ib):
    return (ia ^ (ib * PI2_I32)) & MASK


def _build_body(tab_hbm, grid_hbm, rows, staging, sem):
    wid = lax.axis_index("s") * 2 + lax.axis_index("c")
    lanes = _iota16()
    for l in range(NCOARSE):
        w = W_LVL[l]
        nblk = SH[l] // 16
        cells = w * w
        cbase = wid * np.int32(SH[l])
        copies = []
        for blk in range(nblk):
            c = cbase + (lanes + np.int32(blk * 16))
            c = jnp.minimum(c, np.int32(cells - 1))
            i = c // w
            j = c - i * w
            h = _hash(i, j)
            copies.append(
                pltpu.async_copy(tab_hbm.at[h], rows.at[pl.ds(blk * 16, 16)],
                                 sem)
            )
        for cp in copies:
            cp.wait()
        for blk in range(nblk):
            ridx = lanes + np.int32(blk * 16)
            lc = jnp.full((16,), l, jnp.int32)
            f0 = plsc.load_gather(rows, [ridx, lc])
            f1 = plsc.load_gather(rows, [ridx, lc + L])
            sidx = (lanes + np.int32(blk * 16)) * 2
            plsc.store_scatter(staging, [sidx], f0)
            plsc.store_scatter(staging, [sidx + 1], f1)
        dst = (np.int32(PST[l]) + cbase) * 2
        pltpu.sync_copy(staging.at[pl.ds(0, SH[l] * 2)],
                        grid_hbm.at[pl.ds(dst, SH[l] * 2)])


def _main_body(x0_hbm, x1_hbm, tab_hbm, grid_hbm, out_hbm,
               x0v, x1v, gridv, idxm, rows, outv, sem):
    wid = lax.axis_index("s") * 2 + lax.axis_index("c")
    lanes = _iota16()
    pltpu.sync_copy(grid_hbm, gridv)

    def chunk(g, _):
        base = wid * np.int32(PW) + g * np.int32(C)
        pltpu.sync_copy(x0_hbm.at[pl.ds(base, C)], x0v)
        pltpu.sync_copy(x1_hbm.at[pl.ds(base, C)], x1v)

        for k in range(C // 16):
            xa = x0v[pl.ds(k * 16, 16)]
            xb = x1v[pl.ds(k * 16, 16)]
            for li, l in enumerate(range(NCOARSE, L)):
                nl = jnp.float32(NL[l])
                ia = (xa * nl).astype(jnp.int32)
                ib = (xb * nl).astype(jnp.int32)
                ia1 = ia + 1
                t0 = ib * PI2_I32
                t1 = (ib + 1) * PI2_I32
                hs = (
                    (ia ^ t0) & MASK,
                    (ia1 ^ t0) & MASK,
                    (ia ^ t1) & MASK,
                    (ia1 ^ t1) & MASK,
                )
                for c in range(4):
                    fp = ((k * NFINE + li) * 4 + c) * 16
                    idxm[fp // 128, pl.ds(fp % 128, 16)] = hs[c]

        copies = [
            pltpu.async_copy(
                tab_hbm.at[idxm.at[j]], rows.at[pl.ds(j * 128, 128)], sem
            )
            for j in range(NDMA)
        ]

        for k in range(C // 16):
            xa = x0v[pl.ds(k * 16, 16)]
            xb = x1v[pl.ds(k * 16, 16)]
            pidx = lanes + np.int32(k * 16)
            for l in range(NCOARSE):
                nl = jnp.float32(NL[l])
                w = np.int32(W_LVL[l])
                wa = xa * nl
                wb = xb * nl
                ia = wa.astype(jnp.int32)
                ib = wb.astype(jnp.int32)
                fa = wa - ia.astype(jnp.float32)
                fb = wb - ib.astype(jnp.float32)
                ga = 1.0 - fa
                gb = 1.0 - fb
                w00 = ga * gb
                w10 = fa * gb
                w01 = ga * fb
                w11 = fa * fb
                c00 = (ia * w + ib + np.int32(PST[l])) * 2
                c10 = c00 + 2 * w
                h00f0 = plsc.load_gather(gridv, [c00])
                h00f1 = plsc.load_gather(gridv, [c00 + 1])
                h10f0 = plsc.load_gather(gridv, [c10])
                h10f1 = plsc.load_gather(gridv, [c10 + 1])
                h01f0 = plsc.load_gather(gridv, [c00 + 2])
                h01f1 = plsc.load_gather(gridv, [c00 + 3])
                h11f0 = plsc.load_gather(gridv, [c10 + 2])
                h11f1 = plsc.load_gather(gridv, [c10 + 3])
                y0 = w00 * h00f0 + w10 * h10f0 + w01 * h01f0 + w11 * h11f0
                y1 = w00 * h00f1 + w10 * h10f1 + w01 * h01f1 + w11 * h11f1
                cc = jnp.full((16,), 2 * l, jnp.int32)
                plsc.store_scatter(outv, [pidx, cc], y0)
                plsc.store_scatter(outv, [pidx, cc + 1], y1)

        for cp in copies:
            cp.wait()

        for k in range(C // 16):
            xa = x0v[pl.ds(k * 16, 16)]
            xb = x1v[pl.ds(k * 16, 16)]
            pidx = lanes + np.int32(k * 16)
            for li, l in enumerate(range(NCOARSE, L)):
                nl = jnp.float32(NL[l])
                wa = xa * nl
                wb = xb * nl
                ia = wa.astype(jnp.int32)
                ib = wb.astype(jnp.int32)
                fa = wa - ia.astype(jnp.float32)
                fb = wb - ib.astype(jnp.float32)
                ga = 1.0 - fa
                gb = 1.0 - fb
                w00 = ga * gb
                w10 = fa * gb
                w01 = ga * fb
                w11 = fa * fb
                rbase = (k * NFINE + li) * 64
                lc = jnp.full((16,), l, jnp.int32)
                h = []
                for c in range(4):
                    ridx = lanes + np.int32(rbase + c * 16)
                    h.append(plsc.load_gather(rows, [ridx, lc]))
                    h.append(plsc.load_gather(rows, [ridx, lc + L]))
                y0 = w00 * h[0] + w10 * h[2] + w01 * h[4] + w11 * h[6]
                y1 = w00 * h[1] + w10 * h[3] + w01 * h[5] + w11 * h[7]
                cc = jnp.full((16,), 2 * l, jnp.int32)
                plsc.store_scatter(outv, [pidx, cc], y0)
                plsc.store_scatter(outv, [pidx, cc + 1], y1)

        pltpu.sync_copy(outv, out_hbm.at[pl.ds(base, C), :])
        return ()

    lax.fori_loop(np.int32(0), np.int32(G), chunk, (), unroll=False)


@jax.jit
def _kernel_impl(x, hash_table):
    x0 = x[:, 0].reshape(B)
    x1 = x[:, 1].reshape(B)
    tab = hash_table.reshape(T, F * L)

    build = pl.kernel(
        _build_body,
        out_type=jax.ShapeDtypeStruct((GRID_WORDS,), jnp.float32),
        mesh=plsc.VectorSubcoreMesh(**_MESH),
        scratch_types=[
            pltpu.VMEM((SH_MAX, F * L), jnp.float32),
            pltpu.VMEM((SH_MAX * 2,), jnp.float32),
            pltpu.SemaphoreType.DMA,
        ],
        compiler_params=_SC_PARAMS,
    )
    grid = build(tab)

    run = pl.kernel(
        _main_body,
        out_type=jax.ShapeDtypeStruct((B, F * L), jnp.float32),
        mesh=plsc.VectorSubcoreMesh(**_MESH),
        scratch_types=[
            pltpu.VMEM((C,), jnp.float32),
            pltpu.VMEM((C,), jnp.float32),
            pltpu.VMEM((GRID_WORDS,), jnp.float32),
            pltpu.VMEM((NDMA, 128), jnp.int32),
            pltpu.VMEM((ROWS, F * L), jnp.float32),
            pltpu.VMEM((C, F * L), jnp.float32),
            pltpu.SemaphoreType.DMA,
        ],
        compiler_params=_SC_PARAMS,
    )
    return run(x0, x1, tab, grid)


def kernel(x, hash_table):
    prev = jax.config.jax_enable_x64
    if prev:
        jax.config.update("jax_enable_x64", False)
    try:
        return _kernel_impl(x, hash_table)
    finally:
        if prev:
            jax.config.update("jax_enable_x64", True)
